# Initial kernel scaffold; baseline (speedup 1.0000x reference)
import jax
import jax.numpy as jnp
from jax.experimental import pallas as pl


def kernel(x, edge_index, W1, b1, W2, b2):
    raise NotImplementedError("write your pallas kernel here")


# speedup vs baseline: 21.6405x; 21.6405x over previous
import functools

import jax
import jax.numpy as jnp
from jax import lax
from jax.experimental import pallas as pl
from jax.experimental.pallas import tpu as pltpu
from jax.experimental.pallas import tpu_sc as plsc

N = 10000
E = 320000
D = 128

NC = 2
NS = 16
NW = NC * NS
EPW = E // NW
CH = 125
NCH = EPW // CH
RPS = N // NS
HR = N // 16

_mesh = plsc.VectorSubcoreMesh(core_axis_name="c", subcore_axis_name="s")


def _deg_body(dst_hbm, out_hbm, dst_v, hist_v):
    c = lax.axis_index("c")
    s = lax.axis_index("s")
    w = c * NS + s
    pltpu.sync_copy(dst_hbm.at[w], dst_v)
    zero16 = jnp.zeros((16,), jnp.float32)
    one16 = jnp.ones((16,), jnp.float32)

    def zrow(k, carry):
        hist_v[pl.ds(k * 16, 16)] = zero16
        return carry

    lax.fori_loop(0, N // 16, zrow, 0, unroll=False)

    def acc(k, carry):
        idx = dst_v[k, :]
        plsc.addupdate_scatter(hist_v, [idx], one16)
        return carry

    lax.fori_loop(0, EPW // 16, acc, 0, unroll=False)
    pltpu.sync_copy(hist_v, out_hbm.at[pl.ds(w * N, N)])


_deg = pl.kernel(
    _deg_body,
    out_type=jax.ShapeDtypeStruct((NW * N,), jnp.float32),
    mesh=_mesh,
    scratch_types=[
        pltpu.VMEM((EPW // 16, 16), jnp.int32),
        pltpu.VMEM((N,), jnp.float32),
    ],
    compiler_params=pltpu.CompilerParams(needs_layout_passes=False),
)


def _scat_body(g_hbm, src_hbm, dst_hbm, zeros_hbm, out_hbm,
               src_v, dst_v, rows_v, acc_sh, sem):
    c = lax.axis_index("c")
    s = lax.axis_index("s")
    w = c * NS + s
    pltpu.sync_copy(src_hbm.at[w], src_v)
    pltpu.sync_copy(dst_hbm.at[w], dst_v)
    pltpu.sync_copy(zeros_hbm, acc_sh.at[pl.ds(s * RPS, RPS)])
    plsc.subcore_barrier()

    def chunk(j, carry):
        pltpu.async_copy(g_hbm.at[src_v.at[j]], rows_v, sem).wait()
        pltpu.sync_copy(rows_v, acc_sh.at[dst_v.at[j]], add=True)
        return carry

    lax.fori_loop(0, NCH, chunk, 0, unroll=False)
    plsc.subcore_barrier()
    pltpu.sync_copy(acc_sh.at[pl.ds(s * RPS, RPS)], out_hbm.at[c].at[s])


_scatter = pl.kernel(
    _scat_body,
    out_type=jax.ShapeDtypeStruct((NC, NS, RPS, D), jnp.float32),
    mesh=_mesh,
    scratch_types=[
        pltpu.VMEM((NCH, CH), jnp.int32),
        pltpu.VMEM((NCH, CH), jnp.int32),
        pltpu.VMEM((CH, D), jnp.float32),
        pltpu.VMEM_SHARED((N, D), jnp.float32),
        pltpu.SemaphoreType.DMA,
    ],
)


BLK = 1000
GRID = N // BLK


def _dinv_of(degp_ref):
    deg = 1.0 + jnp.sum(degp_ref[...], axis=1, keepdims=True)
    return lax.rsqrt(deg)


def _mm1_body(x_ref, w_ref, degp_ref, o_ref):
    h = jnp.dot(x_ref[...], w_ref[...], preferred_element_type=jnp.float32)
    o_ref[...] = _dinv_of(degp_ref) * h


_mm1 = pl.pallas_call(
    _mm1_body,
    grid=(GRID,),
    in_specs=[
        pl.BlockSpec((BLK, D), lambda i: (i, 0)),
        pl.BlockSpec((D, D), lambda i: (0, 0)),
        pl.BlockSpec((BLK, NW), lambda i: (i, 0)),
    ],
    out_specs=pl.BlockSpec((BLK, D), lambda i: (i, 0)),
    out_shape=jax.ShapeDtypeStruct((N, D), jnp.float32),
)


def _mid_body(s_ref, g_ref, degp_ref, b_ref, w_ref, o_ref):
    dinv = _dinv_of(degp_ref)
    agg = s_ref[0] + s_ref[1] + g_ref[...]
    z = jnp.maximum(dinv * agg + b_ref[...], 0.0)
    h = jnp.dot(z, w_ref[...], preferred_element_type=jnp.float32)
    o_ref[...] = dinv * h


_mid = pl.pallas_call(
    _mid_body,
    grid=(GRID,),
    in_specs=[
        pl.BlockSpec((NC, BLK, D), lambda i: (0, i, 0)),
        pl.BlockSpec((BLK, D), lambda i: (i, 0)),
        pl.BlockSpec((BLK, NW), lambda i: (i, 0)),
        pl.BlockSpec((1, D), lambda i: (0, 0)),
        pl.BlockSpec((D, D), lambda i: (0, 0)),
    ],
    out_specs=pl.BlockSpec((BLK, D), lambda i: (i, 0)),
    out_shape=jax.ShapeDtypeStruct((N, D), jnp.float32),
)


def _fin_body(s_ref, g_ref, degp_ref, b_ref, o_ref):
    dinv = _dinv_of(degp_ref)
    agg = s_ref[0] + s_ref[1] + g_ref[...]
    o_ref[...] = dinv * agg + b_ref[...]


_fin = pl.pallas_call(
    _fin_body,
    grid=(GRID,),
    in_specs=[
        pl.BlockSpec((NC, BLK, D), lambda i: (0, i, 0)),
        pl.BlockSpec((BLK, D), lambda i: (i, 0)),
        pl.BlockSpec((BLK, NW), lambda i: (i, 0)),
        pl.BlockSpec((1, D), lambda i: (0, 0)),
    ],
    out_specs=pl.BlockSpec((BLK, D), lambda i: (i, 0)),
    out_shape=jax.ShapeDtypeStruct((N, D), jnp.float32),
)


@jax.jit
def kernel(x, edge_index, W1, b1, W2, b2):
    src = edge_index[0].reshape(NW, NCH, CH)
    dst = edge_index[1].reshape(NW, NCH, CH)
    dst16 = edge_index[1].reshape(NW, EPW // 16, 16)
    zeros_r = jnp.zeros((RPS, D), jnp.float32)
    b1r = b1.reshape(1, D)
    b2r = b2.reshape(1, D)

    degp = _deg(dst16).reshape(NW, N).T

    g1 = _mm1(x, W1, degp)
    s1 = _scatter(g1, src, dst, zeros_r).reshape(NC, N, D)
    g2 = _mid(s1, g1, degp, b1r, W2)
    s2 = _scatter(g2, src, dst, zeros_r).reshape(NC, N, D)
    return _fin(s2, g2, degp, b2r)
